# Initial kernel scaffold; baseline (speedup 1.0000x reference)
import jax
import jax.numpy as jnp
from jax.experimental import pallas as pl


def kernel(x, logits, rand_noise, u, targets):
    raise NotImplementedError("write your pallas kernel here")


# speedup vs baseline: 12.0741x; 12.0741x over previous
import functools

import jax
import jax.numpy as jnp
from jax import lax
from jax.experimental import pallas as pl
from jax.experimental.pallas import tpu as pltpu
from jax.experimental.pallas import tpu_sc as plsc

L = 16
NWORKERS = 32


def _sc_body(nrows, xdim, d, noise_hbm, u_hbm, targets_hbm, logits_hbm,
             out_hbm, noise_v, targets_v, u_v, rows_v, stage_v, sem):
    nchunk = xdim // L
    dchunk = d // L
    wid = lax.axis_index("s") * 2 + lax.axis_index("c")
    lanes = lax.iota(jnp.int32, L)
    pltpu.sync_copy(u_hbm, u_v)

    for r in range(nrows):
        b = wid * nrows + r
        pltpu.sync_copy(noise_hbm.at[b], noise_v)
        pltpu.sync_copy(targets_hbm.at[b], targets_v)

        b_vec = jnp.full((L,), b, dtype=jnp.int32)
        u_b = plsc.load_gather(u_v, [b_vec])
        k_vec = jnp.clip((u_b * jnp.float32(xdim)).astype(jnp.int32), 0, xdim - 1)
        k_s = jnp.max(k_vec)

        def count_ge(thr):
            def cbody(i, acc):
                nv = noise_v[pl.ds(pl.multiple_of(i * L, L), L)]
                return acc + jnp.where(nv >= thr, 1, 0)
            acc = lax.fori_loop(0, nchunk, cbody, jnp.zeros((L,), jnp.int32))
            return jnp.sum(acc)

        def bs_body(_, lohi):
            lo, hi = lohi
            mid = (lo + hi + 1) >> 1
            ok = count_ge(mid) >= k_s + 1
            return (jnp.where(ok, mid, lo), jnp.where(ok, hi, mid - 1))

        v_star, _ = lax.fori_loop(0, 12, bs_body,
                                  (jnp.int32(0), jnp.int32(xdim - 1)))

        def gt_body(i, acc):
            nv = noise_v[pl.ds(pl.multiple_of(i * L, L), L)]
            return acc + jnp.where(nv > v_star, 1, 0)
        cnt_gt = jnp.sum(lax.fori_loop(0, nchunk, gt_body,
                                       jnp.zeros((L,), jnp.int32)))
        m = k_s - cnt_gt

        def occ_body(i, carry):
            cum, j_star = carry
            nv = noise_v[pl.ds(pl.multiple_of(i * L, L), L)]
            eq = nv == v_star
            eqi = jnp.where(eq, 1, 0)
            pc = jnp.cumsum(eqi)
            hit = eq & (pc == (m + 1 - cum))
            found = jnp.any(hit)
            lane = jnp.sum(jnp.where(hit, lanes, 0))
            cand = i * L + lane
            j_new = jnp.where((j_star < 0) & found, cand, j_star)
            return (cum + jnp.sum(eqi), j_new)

        _, j_star = lax.fori_loop(0, nchunk, occ_body,
                                  (jnp.int32(0), jnp.int32(-1)))

        rid = b * xdim + j_star
        idx_vec = jnp.full((L,), rid, dtype=jnp.int32)
        pltpu.async_copy(logits_hbm.at[idx_vec], rows_v, sem).wait()

        def mx_body(c, acc):
            ch = rows_v[0, pl.ds(pl.multiple_of(c * L, L), L)]
            return jnp.maximum(acc, ch)
        mx = jnp.max(lax.fori_loop(0, dchunk, mx_body,
                                   jnp.full((L,), -jnp.inf, jnp.float32)))

        def se_body(c, acc):
            ch = rows_v[0, pl.ds(pl.multiple_of(c * L, L), L)]
            return acc + jnp.exp(ch - mx)
        s = jnp.sum(lax.fori_loop(0, dchunk, se_body,
                                  jnp.zeros((L,), jnp.float32)))

        j_vec = jnp.full((L,), j_star, dtype=jnp.int32)
        tgt_vec = plsc.load_gather(targets_v, [j_vec])
        tl_vec = plsc.load_gather(rows_v, [jnp.zeros((L,), jnp.int32), tgt_vec])
        a = jnp.max(tl_vec) - mx

        vec_out = jnp.where(lanes == 0, a, jnp.where(lanes == 1, s, 0.0))
        stage_v[...] = vec_out
        pltpu.sync_copy(stage_v, out_hbm.at[b])


def _tc_body(batch, ab_ref, o_ref):
    a = ab_ref[:, 0:1]
    s = ab_ref[:, 1:2]
    ll = a - jnp.log(s)
    o_ref[0, 0] = jnp.sum(ll) * jnp.float32(1.0 / batch)


def kernel(x, logits, rand_noise, u, targets):
    batch, xdim, d = logits.shape
    nrows = batch // NWORKERS
    mesh = plsc.VectorSubcoreMesh(core_axis_name="c", subcore_axis_name="s")
    sc_call = functools.partial(
        pl.kernel,
        out_type=jax.ShapeDtypeStruct((batch, L), jnp.float32),
        mesh=mesh,
        compiler_params=pltpu.CompilerParams(needs_layout_passes=False),
        scratch_types=[
            pltpu.VMEM((xdim,), jnp.int32),
            pltpu.VMEM((xdim,), jnp.int32),
            pltpu.VMEM((batch,), jnp.float32),
            pltpu.VMEM((L, d), jnp.float32),
            pltpu.VMEM((L,), jnp.float32),
            pltpu.SemaphoreType.DMA,
        ],
    )(functools.partial(_sc_body, nrows, xdim, d))

    ab = sc_call(rand_noise, u, targets.reshape(batch, xdim),
                 logits.reshape(batch * xdim, d))

    out = pl.pallas_call(
        functools.partial(_tc_body, batch),
        out_shape=jax.ShapeDtypeStruct((1, 1), jnp.float32),
        out_specs=pl.BlockSpec(memory_space=pltpu.SMEM),
    )(ab)
    return out[0, 0]


# speedup vs baseline: 17.0895x; 1.4154x over previous
import functools

import jax
import jax.numpy as jnp
from jax import lax
from jax.experimental import pallas as pl
from jax.experimental.pallas import tpu as pltpu
from jax.experimental.pallas import tpu_sc as plsc

L = 16
NWORKERS = 32
UNROLL = 8
INT_MAX = 2147483647


def _zero_hist(hist_v, nchunk):
    zv = jnp.zeros((L,), jnp.int32)

    def z_body(g, carry):
        base = g * (L * UNROLL)
        for t in range(UNROLL):
            hist_v[pl.ds(pl.multiple_of(base + t * L, L), L)] = zv
        return carry

    lax.fori_loop(0, nchunk // UNROLL, z_body, 0)


def _select(noise_v, hist_v, k_s, lanes, lanes_seg, nchunk, seglen):
    nseg = nchunk // (seglen // L)
    del nseg
    ones = jnp.full((L,), 1, jnp.int32)
    kp = jnp.int32(nchunk * L) - k_s

    def h_body(g, carry):
        base = g * (L * UNROLL)
        for t in range(UNROLL):
            nv = noise_v[pl.ds(pl.multiple_of(base + t * L, L), L)]
            plsc.addupdate_scatter(hist_v, [nv], ones)
        return carry

    lax.fori_loop(0, nchunk // UNROLL, h_body, 0)

    def s_body(g, acc):
        i0 = g * UNROLL
        for t in range(UNROLL):
            acc = acc + plsc.load_gather(hist_v, [lanes_seg + (i0 + t)])
        return acc

    seg = lax.fori_loop(0, seglen // UNROLL, s_body,
                        jnp.zeros((L,), jnp.int32))
    cs = plsc.cumsum(seg)
    below = cs < kp
    s_star = jnp.sum(jnp.where(below, 1, 0))
    base_cle = jnp.sum(jnp.where(below, seg, 0))

    seg_base = s_star * seglen
    chs = [hist_v[pl.ds(pl.multiple_of(seg_base + t * L, L), L)]
           for t in range(seglen // L)]
    pcs = [plsc.cumsum(ch) for ch in chs]
    tots = [jnp.max(pc) for pc in pcs]
    run = base_cle
    v_star = jnp.int32(-1)
    cle_v = jnp.int32(0)
    for t in range(seglen // L):
        pc = pcs[t] + run
        hitm = pc >= kp
        lane_cnt = jnp.sum(jnp.where(hitm, 0, 1))
        cand = seg_base + t * L + lane_cnt
        cle_cand = jnp.min(jnp.where(hitm, pc, jnp.int32(INT_MAX)))
        first = (lane_cnt < L) & (v_star < 0)
        v_star = jnp.where(first, cand, v_star)
        cle_v = jnp.where(first, cle_cand, cle_v)
        run = run + tots[t]

    mp1 = cle_v - kp + 1

    def o_body(g, acc):
        i0 = g * UNROLL
        for t in range(UNROLL):
            ng = plsc.load_gather(noise_v, [lanes_seg + (i0 + t)])
            acc = acc + jnp.where(ng == v_star, 1, 0)
        return acc

    seg2 = lax.fori_loop(0, seglen // UNROLL, o_body,
                         jnp.zeros((L,), jnp.int32))
    cs2 = plsc.cumsum(seg2)
    below2 = cs2 < mp1
    p_star = jnp.sum(jnp.where(below2, 1, 0))
    base_occ = jnp.sum(jnp.where(below2, seg2, 0))

    pos_base = p_star * seglen
    nvs = [noise_v[pl.ds(pl.multiple_of(pos_base + t * L, L), L)]
           for t in range(seglen // L)]
    eqs = [jnp.where(nv == v_star, 1, 0) for nv in nvs]
    pcs2 = [plsc.cumsum(eq) for eq in eqs]
    tots2 = [jnp.max(pc) for pc in pcs2]
    run2 = base_occ
    j_star = jnp.int32(-1)
    for t in range(seglen // L):
        pc = pcs2[t] + run2
        hitm = (pc == mp1) & (eqs[t] == 1)
        hit_cnt = jnp.sum(jnp.where(hitm, 1, 0))
        lane = jnp.sum(jnp.where(hitm, lanes, 0))
        first = (hit_cnt > 0) & (j_star < 0)
        j_star = jnp.where(first, pos_base + t * L + lane, j_star)
        run2 = run2 + tots2[t]

    return j_star


def _softmax_stats(rows_v, targets_v, j_star, d):
    dchunk = d // L

    def mx_body(c, acc):
        ch = rows_v[0, pl.ds(pl.multiple_of(c * L, L), L)]
        return jnp.maximum(acc, ch)

    mx = jnp.max(lax.fori_loop(0, dchunk, mx_body,
                               jnp.full((L,), -jnp.inf, jnp.float32)))

    def se_body(c, acc):
        ch = rows_v[0, pl.ds(pl.multiple_of(c * L, L), L)]
        return acc + jnp.exp(ch - mx)

    s = jnp.sum(lax.fori_loop(0, dchunk, se_body,
                              jnp.zeros((L,), jnp.float32)))

    j_vec = jnp.full((L,), j_star, dtype=jnp.int32)
    tgt_vec = plsc.load_gather(targets_v, [j_vec])
    tl_vec = plsc.load_gather(rows_v, [jnp.zeros((L,), jnp.int32), tgt_vec])
    a = jnp.max(tl_vec) - mx
    return a, s


def _sc_body(nrows, xdim, d, noise_hbm, u_hbm, targets_hbm, logits_hbm,
             out_hbm, noise_a, noise_b, tgt_a, tgt_b, u_v, hist_v,
             rows_a, rows_b, stage_v, sem_n0, sem_n1, sem_t0, sem_t1,
             sem_g0, sem_g1):
    nchunk = xdim // L
    seglen = xdim // L
    wid = lax.axis_index("s") * 2 + lax.axis_index("c")
    lanes = lax.iota(jnp.int32, L)
    lanes_seg = lanes * seglen
    b0 = wid * nrows
    b1 = b0 + 1

    h_n0 = pltpu.async_copy(noise_hbm.at[b0], noise_a, sem_n0)
    h_n1 = pltpu.async_copy(noise_hbm.at[b1], noise_b, sem_n1)
    h_t0 = pltpu.async_copy(targets_hbm.at[b0], tgt_a, sem_t0)
    h_t1 = pltpu.async_copy(targets_hbm.at[b1], tgt_b, sem_t1)
    pltpu.sync_copy(u_hbm, u_v)
    _zero_hist(hist_v, nchunk)

    def get_k(b):
        b_vec = jnp.full((L,), b, dtype=jnp.int32)
        u_b = plsc.load_gather(u_v, [b_vec])
        k_vec = jnp.clip((u_b * jnp.float32(xdim)).astype(jnp.int32),
                         0, xdim - 1)
        return jnp.max(k_vec)

    k0 = get_k(b0)
    k1 = get_k(b1)

    h_n0.wait()
    j0 = _select(noise_a, hist_v, k0, lanes, lanes_seg, nchunk, seglen)
    idx0 = jnp.full((L,), b0 * xdim + j0, dtype=jnp.int32)
    h_g0 = pltpu.async_copy(logits_hbm.at[idx0], rows_a, sem_g0)

    _zero_hist(hist_v, nchunk)

    h_n1.wait()
    j1 = _select(noise_b, hist_v, k1, lanes, lanes_seg, nchunk, seglen)
    idx1 = jnp.full((L,), b1 * xdim + j1, dtype=jnp.int32)
    h_g1 = pltpu.async_copy(logits_hbm.at[idx1], rows_b, sem_g1)

    _zero_hist(hist_v, nchunk)

    h_g0.wait()
    h_t0.wait()
    a0, s0 = _softmax_stats(rows_a, tgt_a, j0, d)
    stage_v[...] = jnp.where(lanes == 0, a0, jnp.where(lanes == 1, s0, 0.0))
    pltpu.sync_copy(stage_v, out_hbm.at[b0])

    h_g1.wait()
    h_t1.wait()
    a1, s1 = _softmax_stats(rows_b, tgt_b, j1, d)
    stage_v[...] = jnp.where(lanes == 0, a1, jnp.where(lanes == 1, s1, 0.0))
    pltpu.sync_copy(stage_v, out_hbm.at[b1])


def _tc_body(batch, ab_ref, o_ref):
    a = ab_ref[:, 0:1]
    s = ab_ref[:, 1:2]
    ll = a - jnp.log(s)
    o_ref[0, 0] = jnp.sum(ll) * jnp.float32(1.0 / batch)


def kernel(x, logits, rand_noise, u, targets):
    batch, xdim, d = logits.shape
    nrows = batch // NWORKERS
    mesh = plsc.VectorSubcoreMesh(core_axis_name="c", subcore_axis_name="s")
    sc_call = functools.partial(
        pl.kernel,
        out_type=jax.ShapeDtypeStruct((batch, L), jnp.float32),
        mesh=mesh,
        compiler_params=pltpu.CompilerParams(needs_layout_passes=False),
        scratch_types=[
            pltpu.VMEM((xdim,), jnp.int32),
            pltpu.VMEM((xdim,), jnp.int32),
            pltpu.VMEM((xdim,), jnp.int32),
            pltpu.VMEM((xdim,), jnp.int32),
            pltpu.VMEM((batch,), jnp.float32),
            pltpu.VMEM((xdim,), jnp.int32),
            pltpu.VMEM((L, d), jnp.float32),
            pltpu.VMEM((L, d), jnp.float32),
            pltpu.VMEM((L,), jnp.float32),
            pltpu.SemaphoreType.DMA,
            pltpu.SemaphoreType.DMA,
            pltpu.SemaphoreType.DMA,
            pltpu.SemaphoreType.DMA,
            pltpu.SemaphoreType.DMA,
            pltpu.SemaphoreType.DMA,
        ],
    )(functools.partial(_sc_body, nrows, xdim, d))

    ab = sc_call(rand_noise, u, targets.reshape(batch, xdim),
                 logits.reshape(batch * xdim, d))

    out = pl.pallas_call(
        functools.partial(_tc_body, batch),
        out_shape=jax.ShapeDtypeStruct((1, 1), jnp.float32),
        out_specs=pl.BlockSpec(memory_space=pltpu.SMEM),
    )(ab)
    return out[0, 0]


# speedup vs baseline: 18.0041x; 1.0535x over previous
import functools

import jax
import jax.numpy as jnp
from jax import lax
from jax.experimental import pallas as pl
from jax.experimental.pallas import tpu as pltpu
from jax.experimental.pallas import tpu_sc as plsc

L = 16
NWORKERS = 32
UNROLL = 8
INT_MAX = 2147483647


def _zero_hist(hist_v, nchunk):
    zv = jnp.zeros((L,), jnp.int32)

    def z_body(g, carry):
        base = g * (L * UNROLL)
        for t in range(UNROLL):
            hist_v[pl.ds(pl.multiple_of(base + t * L, L), L)] = zv
        return carry

    lax.fori_loop(0, nchunk // UNROLL, z_body, 0)


def _select(noise_v, hist_v, k_s, lanes, lanes_seg, nchunk, seglen):
    nseg = nchunk // (seglen // L)
    del nseg
    ones = jnp.full((L,), 1, jnp.int32)
    kp = jnp.int32(nchunk * L) - k_s

    def h_body(g, carry):
        base = g * (L * UNROLL)
        nvs = [noise_v[pl.ds(pl.multiple_of(base + t * L, L), L)]
               for t in range(UNROLL)]
        for nv in nvs:
            plsc.addupdate_scatter(hist_v, [nv], ones)
        return carry

    lax.fori_loop(0, nchunk // UNROLL, h_body, 0)

    def s_body(g, acc):
        i0 = g * UNROLL
        for t in range(UNROLL):
            acc = acc + plsc.load_gather(hist_v, [lanes_seg + (i0 + t)])
        return acc

    seg = lax.fori_loop(0, seglen // UNROLL, s_body,
                        jnp.zeros((L,), jnp.int32))
    cs = plsc.cumsum(seg)
    below = cs < kp
    s_star = jnp.sum(jnp.where(below, 1, 0))
    base_cle = jnp.sum(jnp.where(below, seg, 0))

    seg_base = s_star * seglen
    chs = [hist_v[pl.ds(pl.multiple_of(seg_base + t * L, L), L)]
           for t in range(seglen // L)]
    pcs = [plsc.cumsum(ch) for ch in chs]
    tots = [jnp.max(pc) for pc in pcs]
    run = base_cle
    v_star = jnp.int32(-1)
    cle_v = jnp.int32(0)
    for t in range(seglen // L):
        pc = pcs[t] + run
        hitm = pc >= kp
        lane_cnt = jnp.sum(jnp.where(hitm, 0, 1))
        cand = seg_base + t * L + lane_cnt
        cle_cand = jnp.min(jnp.where(hitm, pc, jnp.int32(INT_MAX)))
        first = (lane_cnt < L) & (v_star < 0)
        v_star = jnp.where(first, cand, v_star)
        cle_v = jnp.where(first, cle_cand, cle_v)
        run = run + tots[t]

    mp1 = cle_v - kp + 1

    def o_body(g, acc):
        i0 = g * UNROLL
        for t in range(UNROLL):
            ng = plsc.load_gather(noise_v, [lanes_seg + (i0 + t)])
            acc = acc + jnp.where(ng == v_star, 1, 0)
        return acc

    seg2 = lax.fori_loop(0, seglen // UNROLL, o_body,
                         jnp.zeros((L,), jnp.int32))
    cs2 = plsc.cumsum(seg2)
    below2 = cs2 < mp1
    p_star = jnp.sum(jnp.where(below2, 1, 0))
    base_occ = jnp.sum(jnp.where(below2, seg2, 0))

    pos_base = p_star * seglen
    nvs = [noise_v[pl.ds(pl.multiple_of(pos_base + t * L, L), L)]
           for t in range(seglen // L)]
    eqs = [jnp.where(nv == v_star, 1, 0) for nv in nvs]
    pcs2 = [plsc.cumsum(eq) for eq in eqs]
    tots2 = [jnp.max(pc) for pc in pcs2]
    run2 = base_occ
    j_star = jnp.int32(-1)
    for t in range(seglen // L):
        pc = pcs2[t] + run2
        hitm = (pc == mp1) & (eqs[t] == 1)
        hit_cnt = jnp.sum(jnp.where(hitm, 1, 0))
        lane = jnp.sum(jnp.where(hitm, lanes, 0))
        first = (hit_cnt > 0) & (j_star < 0)
        j_star = jnp.where(first, pos_base + t * L + lane, j_star)
        run2 = run2 + tots2[t]

    return j_star


def _softmax_stats(rows_v, targets_v, j_star, d):
    dchunk = d // L

    def mx_body(c, acc):
        ch = rows_v[0, pl.ds(pl.multiple_of(c * L, L), L)]
        return jnp.maximum(acc, ch)

    mx = jnp.max(lax.fori_loop(0, dchunk, mx_body,
                               jnp.full((L,), -jnp.inf, jnp.float32)))

    def se_body(c, acc):
        ch = rows_v[0, pl.ds(pl.multiple_of(c * L, L), L)]
        return acc + jnp.exp(ch - mx)

    s = jnp.sum(lax.fori_loop(0, dchunk, se_body,
                              jnp.zeros((L,), jnp.float32)))

    j_vec = jnp.full((L,), j_star, dtype=jnp.int32)
    tgt_vec = plsc.load_gather(targets_v, [j_vec])
    tl_vec = plsc.load_gather(rows_v, [jnp.zeros((L,), jnp.int32), tgt_vec])
    a = jnp.max(tl_vec) - mx
    return a, s


def _sc_body(nrows, xdim, d, noise_hbm, u_hbm, targets_hbm, logits_hbm,
             out_hbm, noise_a, noise_b, tgt_a, tgt_b, u_v, hist_v,
             rows_a, rows_b, stage_v, sem_n0, sem_n1, sem_t0, sem_t1,
             sem_g0, sem_g1, sem_u):
    nchunk = xdim // L
    seglen = xdim // L
    wid = lax.axis_index("s") * 2 + lax.axis_index("c")
    lanes = lax.iota(jnp.int32, L)
    lanes_seg = lanes * seglen
    b0 = wid * nrows
    b1 = b0 + 1

    h_n0 = pltpu.async_copy(noise_hbm.at[b0], noise_a, sem_n0)
    h_n1 = pltpu.async_copy(noise_hbm.at[b1], noise_b, sem_n1)
    h_t0 = pltpu.async_copy(targets_hbm.at[b0], tgt_a, sem_t0)
    h_t1 = pltpu.async_copy(targets_hbm.at[b1], tgt_b, sem_t1)
    h_u = pltpu.async_copy(u_hbm, u_v, sem_u)
    _zero_hist(hist_v, nchunk)
    h_u.wait()

    def get_k(b):
        b_vec = jnp.full((L,), b, dtype=jnp.int32)
        u_b = plsc.load_gather(u_v, [b_vec])
        k_vec = jnp.clip((u_b * jnp.float32(xdim)).astype(jnp.int32),
                         0, xdim - 1)
        return jnp.max(k_vec)

    k0 = get_k(b0)
    k1 = get_k(b1)

    h_n0.wait()
    j0 = _select(noise_a, hist_v, k0, lanes, lanes_seg, nchunk, seglen)
    idx0 = jnp.full((L,), b0 * xdim + j0, dtype=jnp.int32)
    h_g0 = pltpu.async_copy(logits_hbm.at[idx0], rows_a, sem_g0)

    _zero_hist(hist_v, nchunk)

    h_n1.wait()
    j1 = _select(noise_b, hist_v, k1, lanes, lanes_seg, nchunk, seglen)
    idx1 = jnp.full((L,), b1 * xdim + j1, dtype=jnp.int32)
    h_g1 = pltpu.async_copy(logits_hbm.at[idx1], rows_b, sem_g1)

    h_g0.wait()
    h_t0.wait()
    a0, s0 = _softmax_stats(rows_a, tgt_a, j0, d)
    stage_v[...] = jnp.where(lanes == 0, a0, jnp.where(lanes == 1, s0, 0.0))
    pltpu.sync_copy(stage_v, out_hbm.at[b0])

    h_g1.wait()
    h_t1.wait()
    a1, s1 = _softmax_stats(rows_b, tgt_b, j1, d)
    stage_v[...] = jnp.where(lanes == 0, a1, jnp.where(lanes == 1, s1, 0.0))
    pltpu.sync_copy(stage_v, out_hbm.at[b1])


def _tc_body(batch, ab_ref, o_ref):
    a = ab_ref[:, 0:1]
    s = ab_ref[:, 1:2]
    ll = a - jnp.log(s)
    o_ref[0, 0] = jnp.sum(ll) * jnp.float32(1.0 / batch)


def kernel(x, logits, rand_noise, u, targets):
    batch, xdim, d = logits.shape
    nrows = batch // NWORKERS
    mesh = plsc.VectorSubcoreMesh(core_axis_name="c", subcore_axis_name="s")
    sc_call = functools.partial(
        pl.kernel,
        out_type=jax.ShapeDtypeStruct((batch, L), jnp.float32),
        mesh=mesh,
        compiler_params=pltpu.CompilerParams(needs_layout_passes=False),
        scratch_types=[
            pltpu.VMEM((xdim,), jnp.int32),
            pltpu.VMEM((xdim,), jnp.int32),
            pltpu.VMEM((xdim,), jnp.int32),
            pltpu.VMEM((xdim,), jnp.int32),
            pltpu.VMEM((batch,), jnp.float32),
            pltpu.VMEM((xdim,), jnp.int32),
            pltpu.VMEM((L, d), jnp.float32),
            pltpu.VMEM((L, d), jnp.float32),
            pltpu.VMEM((L,), jnp.float32),
            pltpu.SemaphoreType.DMA,
            pltpu.SemaphoreType.DMA,
            pltpu.SemaphoreType.DMA,
            pltpu.SemaphoreType.DMA,
            pltpu.SemaphoreType.DMA,
            pltpu.SemaphoreType.DMA,
            pltpu.SemaphoreType.DMA,
        ],
    )(functools.partial(_sc_body, nrows, xdim, d))

    ab = sc_call(rand_noise, u, targets.reshape(batch, xdim),
                 logits.reshape(batch * xdim, d))

    out = pl.pallas_call(
        functools.partial(_tc_body, batch),
        out_shape=jax.ShapeDtypeStruct((1, 1), jnp.float32),
        out_specs=pl.BlockSpec(memory_space=pltpu.SMEM),
    )(ab)
    return out[0, 0]


# speedup vs baseline: 18.3279x; 1.0180x over previous
import functools

import jax
import jax.numpy as jnp
from jax import lax
from jax.experimental import pallas as pl
from jax.experimental.pallas import tpu as pltpu
from jax.experimental.pallas import tpu_sc as plsc

L = 16
NWORKERS = 32
UNROLL = 8
INT_MAX = 2147483647


def _zero_hist(hist_v, nchunk):
    zv = jnp.zeros((L,), jnp.int32)

    def z_body(g, carry):
        base = g * (L * UNROLL)
        for t in range(UNROLL):
            hist_v[pl.ds(pl.multiple_of(base + t * L, L), L)] = zv
        return carry

    lax.fori_loop(0, nchunk // UNROLL, z_body, 0)


def _select(noise_v, hist_v, k_s, lanes, lanes_seg, nchunk, seglen):
    ones = jnp.full((L,), 1, jnp.int32)
    zv = jnp.zeros((L,), jnp.int32)
    kp = jnp.int32(nchunk * L) - k_s

    def h_body(g, carry):
        base = g * (L * UNROLL)
        nvs = [noise_v[pl.ds(pl.multiple_of(base + t * L, L), L)]
               for t in range(UNROLL)]
        for nv in nvs:
            plsc.addupdate_scatter(hist_v, [nv], ones)
        return carry

    lax.fori_loop(0, nchunk // UNROLL, h_body, 0)

    def s_body(g, acc):
        i0 = g * UNROLL
        for t in range(UNROLL):
            acc = acc + plsc.load_gather(hist_v, [lanes_seg + (i0 + t)])
        return acc

    seg = lax.fori_loop(0, seglen // UNROLL, s_body,
                        jnp.zeros((L,), jnp.int32))
    cs = plsc.cumsum(seg)
    below = cs < kp
    s_star = jnp.sum(jnp.where(below, 1, 0))
    base_cle = jnp.sum(jnp.where(below, seg, 0))

    seg_base = s_star * seglen
    chs = [hist_v[pl.ds(pl.multiple_of(seg_base + t * L, L), L)]
           for t in range(seglen // L)]
    pcs = [plsc.cumsum(ch) for ch in chs]
    tots = [jnp.max(pc) for pc in pcs]
    run = base_cle
    v_star = jnp.int32(-1)
    cle_v = jnp.int32(0)
    for t in range(seglen // L):
        pc = pcs[t] + run
        hitm = pc >= kp
        lane_cnt = jnp.sum(jnp.where(hitm, 0, 1))
        cand = seg_base + t * L + lane_cnt
        cle_cand = jnp.min(jnp.where(hitm, pc, jnp.int32(INT_MAX)))
        first = (lane_cnt < L) & (v_star < 0)
        v_star = jnp.where(first, cand, v_star)
        cle_v = jnp.where(first, cle_cand, cle_v)
        run = run + tots[t]

    mp1 = cle_v - kp + 1

    def o_body(g, acc):
        i0 = g * UNROLL
        for t in range(UNROLL):
            ng = plsc.load_gather(noise_v, [lanes_seg + (i0 + t)])
            acc = acc + jnp.where(ng == v_star, 1, 0)
            hist_v[pl.ds(pl.multiple_of((g * UNROLL + t) * L, L), L)] = zv
        return acc

    seg2 = lax.fori_loop(0, seglen // UNROLL, o_body,
                         jnp.zeros((L,), jnp.int32))
    cs2 = plsc.cumsum(seg2)
    below2 = cs2 < mp1
    p_star = jnp.sum(jnp.where(below2, 1, 0))
    base_occ = jnp.sum(jnp.where(below2, seg2, 0))

    pos_base = p_star * seglen
    nvs = [noise_v[pl.ds(pl.multiple_of(pos_base + t * L, L), L)]
           for t in range(seglen // L)]
    eqs = [jnp.where(nv == v_star, 1, 0) for nv in nvs]
    pcs2 = [plsc.cumsum(eq) for eq in eqs]
    tots2 = [jnp.max(pc) for pc in pcs2]
    run2 = base_occ
    j_star = jnp.int32(-1)
    for t in range(seglen // L):
        pc = pcs2[t] + run2
        hitm = (pc == mp1) & (eqs[t] == 1)
        hit_cnt = jnp.sum(jnp.where(hitm, 1, 0))
        lane = jnp.sum(jnp.where(hitm, lanes, 0))
        first = (hit_cnt > 0) & (j_star < 0)
        j_star = jnp.where(first, pos_base + t * L + lane, j_star)
        run2 = run2 + tots2[t]

    return j_star


def _softmax_stats(rows_v, targets_v, j_star, b, d):
    dchunk = d // L

    def mx_body(c, acc):
        ch = rows_v[0, pl.ds(pl.multiple_of(c * L, L), L)]
        return jnp.maximum(acc, ch)

    mx = jnp.max(lax.fori_loop(0, dchunk, mx_body,
                               jnp.full((L,), -jnp.inf, jnp.float32)))

    def se_body(c, acc):
        ch = rows_v[0, pl.ds(pl.multiple_of(c * L, L), L)]
        return acc + jnp.exp(ch - mx)

    s = jnp.sum(lax.fori_loop(0, dchunk, se_body,
                              jnp.zeros((L,), jnp.float32)))

    wv = jnp.full((L,), j_star & 31, dtype=jnp.int32)
    bv = jnp.full((L,), b, dtype=jnp.int32)
    tgt_vec = plsc.load_gather(targets_v, [wv, bv])
    tl_vec = plsc.load_gather(rows_v, [jnp.zeros((L,), jnp.int32), tgt_vec])
    a = jnp.max(tl_vec) - mx
    return a, s


def _sc_body(nrows, xdim, d, noise_hbm, u_hbm, targets_hbm, logits_hbm,
             out_hbm, noise_a, noise_b, u_v, hist_v, tgt_a, tgt_b,
             rows_a, rows_b, stage_a, stage_b, sem_n0, sem_n1, sem_t0,
             sem_t1, sem_g0, sem_g1, sem_u, sem_o0, sem_o1):
    nchunk = xdim // L
    seglen = xdim // L
    wid = lax.axis_index("s") * 2 + lax.axis_index("c")
    lanes = lax.iota(jnp.int32, L)
    lanes_seg = lanes * seglen
    b0 = wid * nrows
    b1 = b0 + 1

    h_n0 = pltpu.async_copy(noise_hbm.at[b0], noise_a, sem_n0)
    h_n1 = pltpu.async_copy(noise_hbm.at[b1], noise_b, sem_n1)
    h_u = pltpu.async_copy(u_hbm, u_v, sem_u)
    _zero_hist(hist_v, nchunk)
    h_u.wait()

    def get_k(b):
        b_vec = jnp.full((L,), b, dtype=jnp.int32)
        u_b = plsc.load_gather(u_v, [b_vec])
        k_vec = jnp.clip((u_b * jnp.float32(xdim)).astype(jnp.int32),
                         0, xdim - 1)
        return jnp.max(k_vec)

    k0 = get_k(b0)
    k1 = get_k(b1)

    h_n0.wait()
    j0 = _select(noise_a, hist_v, k0, lanes, lanes_seg, nchunk, seglen)
    idx0 = jnp.full((L,), b0 * xdim + j0, dtype=jnp.int32)
    h_g0 = pltpu.async_copy(logits_hbm.at[idx0], rows_a, sem_g0)
    h_t0 = pltpu.async_copy(targets_hbm.at[j0 >> 10, (j0 >> 5) & 31],
                            tgt_a, sem_t0)

    h_n1.wait()
    j1 = _select(noise_b, hist_v, k1, lanes, lanes_seg, nchunk, seglen)
    idx1 = jnp.full((L,), b1 * xdim + j1, dtype=jnp.int32)
    h_g1 = pltpu.async_copy(logits_hbm.at[idx1], rows_b, sem_g1)
    h_t1 = pltpu.async_copy(targets_hbm.at[j1 >> 10, (j1 >> 5) & 31],
                            tgt_b, sem_t1)

    h_g0.wait()
    h_t0.wait()
    a0, s0 = _softmax_stats(rows_a, tgt_a, j0, b0, d)
    stage_a[...] = jnp.where(lanes == 0, a0, jnp.where(lanes == 1, s0, 0.0))
    h_o0 = pltpu.async_copy(stage_a, out_hbm.at[b0], sem_o0)

    h_g1.wait()
    h_t1.wait()
    a1, s1 = _softmax_stats(rows_b, tgt_b, j1, b1, d)
    stage_b[...] = jnp.where(lanes == 0, a1, jnp.where(lanes == 1, s1, 0.0))
    h_o1 = pltpu.async_copy(stage_b, out_hbm.at[b1], sem_o1)

    h_o0.wait()
    h_o1.wait()


def _tc_body(batch, ab_ref, o_ref):
    a = ab_ref[:, 0:1]
    s = ab_ref[:, 1:2]
    ll = a - jnp.log(s)
    o_ref[0, 0] = jnp.sum(ll) * jnp.float32(1.0 / batch)


def kernel(x, logits, rand_noise, u, targets):
    batch, xdim, d = logits.shape
    nrows = batch // NWORKERS
    mesh = plsc.VectorSubcoreMesh(core_axis_name="c", subcore_axis_name="s")
    sc_call = functools.partial(
        pl.kernel,
        out_type=jax.ShapeDtypeStruct((batch, L), jnp.float32),
        mesh=mesh,
        compiler_params=pltpu.CompilerParams(needs_layout_passes=False),
        scratch_types=[
            pltpu.VMEM((xdim,), jnp.int32),
            pltpu.VMEM((xdim,), jnp.int32),
            pltpu.VMEM((batch,), jnp.float32),
            pltpu.VMEM((xdim,), jnp.int32),
            pltpu.VMEM((32, batch), jnp.int32),
            pltpu.VMEM((32, batch), jnp.int32),
            pltpu.VMEM((L, d), jnp.float32),
            pltpu.VMEM((L, d), jnp.float32),
            pltpu.VMEM((L,), jnp.float32),
            pltpu.VMEM((L,), jnp.float32),
            pltpu.SemaphoreType.DMA,
            pltpu.SemaphoreType.DMA,
            pltpu.SemaphoreType.DMA,
            pltpu.SemaphoreType.DMA,
            pltpu.SemaphoreType.DMA,
            pltpu.SemaphoreType.DMA,
            pltpu.SemaphoreType.DMA,
            pltpu.SemaphoreType.DMA,
            pltpu.SemaphoreType.DMA,
        ],
    )(functools.partial(_sc_body, nrows, xdim, d))

    tgt_t = jnp.transpose(targets, (1, 2, 3, 0))
    ab = sc_call(rand_noise, u, tgt_t, logits.reshape(batch * xdim, d))

    out = pl.pallas_call(
        functools.partial(_tc_body, batch),
        out_shape=jax.ShapeDtypeStruct((1, 1), jnp.float32),
        out_specs=pl.BlockSpec(memory_space=pltpu.SMEM),
    )(ab)
    return out[0, 0]


# speedup vs baseline: 20.6293x; 1.1256x over previous
import functools

import jax
import jax.numpy as jnp
from jax import lax
from jax.experimental import pallas as pl
from jax.experimental.pallas import tpu as pltpu
from jax.experimental.pallas import tpu_sc as plsc

L = 16
NWORKERS = 32
UNROLL = 8
INT_MAX = 2147483647


def _zero_hist(hist_v, nchunk):
    zv = jnp.zeros((L,), jnp.int32)

    def z_body(g, carry):
        base = g * (L * UNROLL)
        for t in range(UNROLL):
            hist_v[pl.ds(pl.multiple_of(base + t * L, L), L)] = zv
        return carry

    lax.fori_loop(0, nchunk // UNROLL, z_body, 0)


def _select(noise_v, hist_v, k_s, lanes, lanes_seg, nchunk, seglen):
    ones = jnp.full((L,), 1, jnp.int32)
    zv = jnp.zeros((L,), jnp.int32)
    kp = jnp.int32(nchunk * L) - k_s

    def h_body(g, carry):
        base = g * (L * UNROLL)
        nvs = [noise_v[pl.ds(pl.multiple_of(base + t * L, L), L)]
               for t in range(UNROLL)]
        for nv in nvs:
            plsc.addupdate_scatter(hist_v, [nv], ones)
        return carry

    lax.fori_loop(0, nchunk // UNROLL, h_body, 0)

    seglen_c = jnp.int32(seglen)

    def s_body(g, acc):
        i0 = g * UNROLL
        for t in range(UNROLL):
            w = lanes + (i0 + t)
            w = jnp.where(w >= seglen_c, w - seglen_c, w)
            acc = acc + plsc.load_gather(hist_v, [lanes_seg + w])
        return acc

    seg = lax.fori_loop(0, seglen // UNROLL, s_body,
                        jnp.zeros((L,), jnp.int32))
    cs = plsc.cumsum(seg)
    below = cs < kp
    s_star = jnp.sum(jnp.where(below, 1, 0))
    base_cle = jnp.sum(jnp.where(below, seg, 0))

    seg_base = s_star * seglen
    chs = [hist_v[pl.ds(pl.multiple_of(seg_base + t * L, L), L)]
           for t in range(seglen // L)]
    pcs = [plsc.cumsum(ch) for ch in chs]
    tots = [jnp.max(pc) for pc in pcs]
    run = base_cle
    v_star = jnp.int32(-1)
    cle_v = jnp.int32(0)
    for t in range(seglen // L):
        pc = pcs[t] + run
        hitm = pc >= kp
        lane_cnt = jnp.sum(jnp.where(hitm, 0, 1))
        cand = seg_base + t * L + lane_cnt
        cle_cand = jnp.min(jnp.where(hitm, pc, jnp.int32(INT_MAX)))
        first = (lane_cnt < L) & (v_star < 0)
        v_star = jnp.where(first, cand, v_star)
        cle_v = jnp.where(first, cle_cand, cle_v)
        run = run + tots[t]

    mp1 = cle_v - kp + 1

    def o_body(g, acc):
        i0 = g * UNROLL
        for t in range(UNROLL):
            w = lanes + (i0 + t)
            w = jnp.where(w >= seglen_c, w - seglen_c, w)
            ng = plsc.load_gather(noise_v, [lanes_seg + w])
            acc = acc + jnp.where(ng == v_star, 1, 0)
            hist_v[pl.ds(pl.multiple_of((g * UNROLL + t) * L, L), L)] = zv
        return acc

    seg2 = lax.fori_loop(0, seglen // UNROLL, o_body,
                         jnp.zeros((L,), jnp.int32))
    cs2 = plsc.cumsum(seg2)
    below2 = cs2 < mp1
    p_star = jnp.sum(jnp.where(below2, 1, 0))
    base_occ = jnp.sum(jnp.where(below2, seg2, 0))

    pos_base = p_star * seglen
    nvs = [noise_v[pl.ds(pl.multiple_of(pos_base + t * L, L), L)]
           for t in range(seglen // L)]
    eqs = [jnp.where(nv == v_star, 1, 0) for nv in nvs]
    pcs2 = [plsc.cumsum(eq) for eq in eqs]
    tots2 = [jnp.max(pc) for pc in pcs2]
    run2 = base_occ
    j_star = jnp.int32(-1)
    for t in range(seglen // L):
        pc = pcs2[t] + run2
        hitm = (pc == mp1) & (eqs[t] == 1)
        hit_cnt = jnp.sum(jnp.where(hitm, 1, 0))
        lane = jnp.sum(jnp.where(hitm, lanes, 0))
        first = (hit_cnt > 0) & (j_star < 0)
        j_star = jnp.where(first, pos_base + t * L + lane, j_star)
        run2 = run2 + tots2[t]

    return j_star


def _softmax_stats(rows_v, targets_v, j_star, b, d):
    dchunk = d // L

    def mx_body(c, acc):
        ch = rows_v[0, pl.ds(pl.multiple_of(c * L, L), L)]
        return jnp.maximum(acc, ch)

    mx = jnp.max(lax.fori_loop(0, dchunk, mx_body,
                               jnp.full((L,), -jnp.inf, jnp.float32)))

    def se_body(c, acc):
        ch = rows_v[0, pl.ds(pl.multiple_of(c * L, L), L)]
        return acc + jnp.exp(ch - mx)

    s = jnp.sum(lax.fori_loop(0, dchunk, se_body,
                              jnp.zeros((L,), jnp.float32)))

    wv = jnp.full((L,), j_star & 31, dtype=jnp.int32)
    bv = jnp.full((L,), b, dtype=jnp.int32)
    tgt_vec = plsc.load_gather(targets_v, [wv, bv])
    tl_vec = plsc.load_gather(rows_v, [jnp.zeros((L,), jnp.int32), tgt_vec])
    a = jnp.max(tl_vec) - mx
    return a, s


def _sc_body(nrows, xdim, d, noise_hbm, u_hbm, targets_hbm, logits_hbm,
             out_hbm, noise_a, noise_b, u_v, hist_v, tgt_a, tgt_b,
             rows_a, rows_b, stage_a, stage_b, sem_n0, sem_n1, sem_t0,
             sem_t1, sem_g0, sem_g1, sem_u, sem_o0, sem_o1):
    nchunk = xdim // L
    seglen = xdim // L
    wid = lax.axis_index("s") * 2 + lax.axis_index("c")
    lanes = lax.iota(jnp.int32, L)
    lanes_seg = lanes * seglen
    b0 = wid * nrows
    b1 = b0 + 1

    h_n0 = pltpu.async_copy(noise_hbm.at[b0], noise_a, sem_n0)
    h_n1 = pltpu.async_copy(noise_hbm.at[b1], noise_b, sem_n1)
    h_u = pltpu.async_copy(u_hbm, u_v, sem_u)
    _zero_hist(hist_v, nchunk)
    h_u.wait()

    def get_k(b):
        b_vec = jnp.full((L,), b, dtype=jnp.int32)
        u_b = plsc.load_gather(u_v, [b_vec])
        k_vec = jnp.clip((u_b * jnp.float32(xdim)).astype(jnp.int32),
                         0, xdim - 1)
        return jnp.max(k_vec)

    k0 = get_k(b0)
    k1 = get_k(b1)

    h_n0.wait()
    j0 = _select(noise_a, hist_v, k0, lanes, lanes_seg, nchunk, seglen)
    idx0 = jnp.full((L,), b0 * xdim + j0, dtype=jnp.int32)
    h_g0 = pltpu.async_copy(logits_hbm.at[idx0], rows_a, sem_g0)
    h_t0 = pltpu.async_copy(targets_hbm.at[j0 >> 10, (j0 >> 5) & 31],
                            tgt_a, sem_t0)

    h_n1.wait()
    j1 = _select(noise_b, hist_v, k1, lanes, lanes_seg, nchunk, seglen)
    idx1 = jnp.full((L,), b1 * xdim + j1, dtype=jnp.int32)
    h_g1 = pltpu.async_copy(logits_hbm.at[idx1], rows_b, sem_g1)
    h_t1 = pltpu.async_copy(targets_hbm.at[j1 >> 10, (j1 >> 5) & 31],
                            tgt_b, sem_t1)

    h_g0.wait()
    h_t0.wait()
    a0, s0 = _softmax_stats(rows_a, tgt_a, j0, b0, d)
    stage_a[...] = jnp.where(lanes == 0, a0, jnp.where(lanes == 1, s0, 0.0))
    h_o0 = pltpu.async_copy(stage_a, out_hbm.at[b0], sem_o0)

    h_g1.wait()
    h_t1.wait()
    a1, s1 = _softmax_stats(rows_b, tgt_b, j1, b1, d)
    stage_b[...] = jnp.where(lanes == 0, a1, jnp.where(lanes == 1, s1, 0.0))
    h_o1 = pltpu.async_copy(stage_b, out_hbm.at[b1], sem_o1)

    h_o0.wait()
    h_o1.wait()


def _tc_body(batch, ab_ref, o_ref):
    a = ab_ref[:, 0:1]
    s = ab_ref[:, 1:2]
    ll = a - jnp.log(s)
    o_ref[0, 0] = jnp.sum(ll) * jnp.float32(1.0 / batch)


def kernel(x, logits, rand_noise, u, targets):
    batch, xdim, d = logits.shape
    nrows = batch // NWORKERS
    mesh = plsc.VectorSubcoreMesh(core_axis_name="c", subcore_axis_name="s")
    sc_call = functools.partial(
        pl.kernel,
        out_type=jax.ShapeDtypeStruct((batch, L), jnp.float32),
        mesh=mesh,
        compiler_params=pltpu.CompilerParams(needs_layout_passes=False),
        scratch_types=[
            pltpu.VMEM((xdim,), jnp.int32),
            pltpu.VMEM((xdim,), jnp.int32),
            pltpu.VMEM((batch,), jnp.float32),
            pltpu.VMEM((xdim,), jnp.int32),
            pltpu.VMEM((32, batch), jnp.int32),
            pltpu.VMEM((32, batch), jnp.int32),
            pltpu.VMEM((L, d), jnp.float32),
            pltpu.VMEM((L, d), jnp.float32),
            pltpu.VMEM((L,), jnp.float32),
            pltpu.VMEM((L,), jnp.float32),
            pltpu.SemaphoreType.DMA,
            pltpu.SemaphoreType.DMA,
            pltpu.SemaphoreType.DMA,
            pltpu.SemaphoreType.DMA,
            pltpu.SemaphoreType.DMA,
            pltpu.SemaphoreType.DMA,
            pltpu.SemaphoreType.DMA,
            pltpu.SemaphoreType.DMA,
            pltpu.SemaphoreType.DMA,
        ],
    )(functools.partial(_sc_body, nrows, xdim, d))

    tgt_t = jnp.transpose(targets, (1, 2, 3, 0))
    ab = sc_call(rand_noise, u, tgt_t, logits.reshape(batch * xdim, d))

    out = pl.pallas_call(
        functools.partial(_tc_body, batch),
        out_shape=jax.ShapeDtypeStruct((1, 1), jnp.float32),
        out_specs=pl.BlockSpec(memory_space=pltpu.SMEM),
    )(ab)
    return out[0, 0]


# speedup vs baseline: 21.4499x; 1.0398x over previous
import functools

import jax
import jax.numpy as jnp
from jax import lax
from jax.experimental import pallas as pl
from jax.experimental.pallas import tpu as pltpu
from jax.experimental.pallas import tpu_sc as plsc

L = 16
NWORKERS = 32
UNROLL = 8
INT_MAX = 2147483647


def _zero_hist(hist_v, nchunk):
    zv = jnp.zeros((L,), jnp.int32)

    def z_body(g, carry):
        base = g * (L * UNROLL)
        for t in range(UNROLL):
            hist_v[pl.ds(pl.multiple_of(base + t * L, L), L)] = zv
        return carry

    lax.fori_loop(0, nchunk // UNROLL, z_body, 0)


def _select(noise_v, hist_v, k_s, lanes, lanes_seg, nchunk, seglen):
    ones = jnp.full((L,), 1, jnp.int32)
    zv = jnp.zeros((L,), jnp.int32)
    kp = jnp.int32(nchunk * L) - k_s

    def h_body(g, carry):
        base = g * (L * UNROLL)
        nvs = [noise_v[pl.ds(pl.multiple_of(base + t * L, L), L)]
               for t in range(UNROLL)]
        for nv in nvs:
            plsc.addupdate_scatter(hist_v, [nv], ones)
        return carry

    lax.fori_loop(0, nchunk // UNROLL, h_body, 0)

    seglen_c = jnp.int32(seglen)

    def s_body(g, acc):
        i0 = g * UNROLL
        for t in range(UNROLL):
            w = lanes + (i0 + t)
            w = jnp.where(w >= seglen_c, w - seglen_c, w)
            acc = acc + plsc.load_gather(hist_v, [lanes_seg + w])
        return acc

    seg = lax.fori_loop(0, seglen // UNROLL, s_body,
                        jnp.zeros((L,), jnp.int32))
    cs = plsc.cumsum(seg)
    below = cs < kp
    s_star = jnp.sum(jnp.where(below, 1, 0))
    base_cle = jnp.sum(jnp.where(below, seg, 0))

    seg_base = s_star * seglen
    chs = [hist_v[pl.ds(pl.multiple_of(seg_base + t * L, L), L)]
           for t in range(seglen // L)]
    pcs = [plsc.cumsum(ch) for ch in chs]
    tots = [jnp.max(pc) for pc in pcs]
    run = base_cle
    v_star = jnp.int32(-1)
    cle_v = jnp.int32(0)
    for t in range(seglen // L):
        pc = pcs[t] + run
        hitm = pc >= kp
        lane_cnt = jnp.sum(jnp.where(hitm, 0, 1))
        cand = seg_base + t * L + lane_cnt
        cle_cand = jnp.min(jnp.where(hitm, pc, jnp.int32(INT_MAX)))
        first = (lane_cnt < L) & (v_star < 0)
        v_star = jnp.where(first, cand, v_star)
        cle_v = jnp.where(first, cle_cand, cle_v)
        run = run + tots[t]

    mp1 = cle_v - kp + 1

    def o_body(g, acc):
        i0 = g * UNROLL
        for t in range(UNROLL):
            w = lanes + (i0 + t)
            w = jnp.where(w >= seglen_c, w - seglen_c, w)
            ng = plsc.load_gather(noise_v, [lanes_seg + w])
            acc = acc + jnp.where(ng == v_star, 1, 0)
            hist_v[pl.ds(pl.multiple_of((g * UNROLL + t) * L, L), L)] = zv
        return acc

    seg2 = lax.fori_loop(0, seglen // UNROLL, o_body,
                         jnp.zeros((L,), jnp.int32))
    cs2 = plsc.cumsum(seg2)
    below2 = cs2 < mp1
    p_star = jnp.sum(jnp.where(below2, 1, 0))
    base_occ = jnp.sum(jnp.where(below2, seg2, 0))

    pos_base = p_star * seglen
    nvs = [noise_v[pl.ds(pl.multiple_of(pos_base + t * L, L), L)]
           for t in range(seglen // L)]
    eqs = [jnp.where(nv == v_star, 1, 0) for nv in nvs]
    pcs2 = [plsc.cumsum(eq) for eq in eqs]
    tots2 = [jnp.max(pc) for pc in pcs2]
    run2 = base_occ
    j_star = jnp.int32(-1)
    for t in range(seglen // L):
        pc = pcs2[t] + run2
        hitm = (pc == mp1) & (eqs[t] == 1)
        hit_cnt = jnp.sum(jnp.where(hitm, 1, 0))
        lane = jnp.sum(jnp.where(hitm, lanes, 0))
        first = (hit_cnt > 0) & (j_star < 0)
        j_star = jnp.where(first, pos_base + t * L + lane, j_star)
        run2 = run2 + tots2[t]

    return j_star


def _softmax_stats(rows_v, targets_v, j_star, b, d):
    dchunk = d // L

    chs = [rows_v[0, pl.ds(pl.multiple_of(c * L, L), L)]
           for c in range(dchunk)]
    ms = chs
    while len(ms) > 1:
        ms = [jnp.maximum(ms[i], ms[i + 1]) for i in range(0, len(ms) - 1, 2)] \
            + ([ms[-1]] if len(ms) % 2 else [])
    mx = jnp.max(ms[0])
    es = [jnp.exp(ch - mx) for ch in chs]
    while len(es) > 1:
        es = [es[i] + es[i + 1] for i in range(0, len(es) - 1, 2)] \
            + ([es[-1]] if len(es) % 2 else [])
    s = jnp.sum(es[0])

    wv = jnp.full((L,), j_star & 31, dtype=jnp.int32)
    bv = jnp.full((L,), b, dtype=jnp.int32)
    tgt_vec = plsc.load_gather(targets_v, [wv, bv])
    tl_vec = plsc.load_gather(rows_v, [jnp.zeros((L,), jnp.int32), tgt_vec])
    a = jnp.max(tl_vec) - mx
    return a, s


def _sc_body(nrows, xdim, d, noise_hbm, u_hbm, targets_hbm, logits_hbm,
             out_hbm, noise_a, noise_b, u_v, hist_v, tgt_a, tgt_b,
             rows_a, rows_b, idx_a, idx_b, stage_a, stage_b, sem_n0,
             sem_n1, sem_t0, sem_t1, sem_g0, sem_g1, sem_u, sem_o0,
             sem_o1):
    nchunk = xdim // L
    seglen = xdim // L
    wid = lax.axis_index("s") * 2 + lax.axis_index("c")
    lanes = lax.iota(jnp.int32, L)
    lanes_seg = lanes * seglen
    b0 = wid * nrows
    b1 = b0 + 1

    h_n0 = pltpu.async_copy(noise_hbm.at[b0], noise_a, sem_n0)
    h_n1 = pltpu.async_copy(noise_hbm.at[b1], noise_b, sem_n1)
    h_u = pltpu.async_copy(u_hbm, u_v, sem_u)
    _zero_hist(hist_v, nchunk)
    h_u.wait()

    def get_k(b):
        b_vec = jnp.full((L,), b, dtype=jnp.int32)
        u_b = plsc.load_gather(u_v, [b_vec])
        k_vec = jnp.clip((u_b * jnp.float32(xdim)).astype(jnp.int32),
                         0, xdim - 1)
        return jnp.max(k_vec)

    k0 = get_k(b0)
    k1 = get_k(b1)

    two = lanes < 2
    idx_lanes = jnp.where(two, lanes, 0)

    h_n0.wait()
    j0 = _select(noise_a, hist_v, k0, lanes, lanes_seg, nchunk, seglen)
    plsc.store_scatter(idx_a, [idx_lanes],
                       jnp.full((L,), b0 * xdim + j0, dtype=jnp.int32),
                       mask=two)
    h_g0 = pltpu.async_copy(logits_hbm.at[idx_a], rows_a, sem_g0)
    h_t0 = pltpu.async_copy(targets_hbm.at[j0 >> 10, (j0 >> 5) & 31],
                            tgt_a, sem_t0)

    h_n1.wait()
    j1 = _select(noise_b, hist_v, k1, lanes, lanes_seg, nchunk, seglen)
    plsc.store_scatter(idx_b, [idx_lanes],
                       jnp.full((L,), b1 * xdim + j1, dtype=jnp.int32),
                       mask=two)
    h_g1 = pltpu.async_copy(logits_hbm.at[idx_b], rows_b, sem_g1)
    h_t1 = pltpu.async_copy(targets_hbm.at[j1 >> 10, (j1 >> 5) & 31],
                            tgt_b, sem_t1)

    h_g0.wait()
    h_t0.wait()
    a0, s0 = _softmax_stats(rows_a, tgt_a, j0, b0, d)
    stage_a[...] = jnp.where(lanes == 0, a0, jnp.where(lanes == 1, s0, 0.0))
    h_o0 = pltpu.async_copy(stage_a, out_hbm.at[b0], sem_o0)

    h_g1.wait()
    h_t1.wait()
    a1, s1 = _softmax_stats(rows_b, tgt_b, j1, b1, d)
    stage_b[...] = jnp.where(lanes == 0, a1, jnp.where(lanes == 1, s1, 0.0))
    h_o1 = pltpu.async_copy(stage_b, out_hbm.at[b1], sem_o1)

    h_o0.wait()
    h_o1.wait()


def _tc_body(batch, ab_ref, o_ref):
    a = ab_ref[:, 0:1]
    s = ab_ref[:, 1:2]
    ll = a - jnp.log(s)
    o_ref[0, 0] = jnp.sum(ll) * jnp.float32(1.0 / batch)


def kernel(x, logits, rand_noise, u, targets):
    batch, xdim, d = logits.shape
    nrows = batch // NWORKERS
    mesh = plsc.VectorSubcoreMesh(core_axis_name="c", subcore_axis_name="s")
    sc_call = functools.partial(
        pl.kernel,
        out_type=jax.ShapeDtypeStruct((batch, L), jnp.float32),
        mesh=mesh,
        compiler_params=pltpu.CompilerParams(needs_layout_passes=False),
        scratch_types=[
            pltpu.VMEM((xdim,), jnp.int32),
            pltpu.VMEM((xdim,), jnp.int32),
            pltpu.VMEM((batch,), jnp.float32),
            pltpu.VMEM((xdim,), jnp.int32),
            pltpu.VMEM((32, batch), jnp.int32),
            pltpu.VMEM((32, batch), jnp.int32),
            pltpu.VMEM((2, d), jnp.float32),
            pltpu.VMEM((2, d), jnp.float32),
            pltpu.VMEM((2,), jnp.int32),
            pltpu.VMEM((2,), jnp.int32),
            pltpu.VMEM((L,), jnp.float32),
            pltpu.VMEM((L,), jnp.float32),
            pltpu.SemaphoreType.DMA,
            pltpu.SemaphoreType.DMA,
            pltpu.SemaphoreType.DMA,
            pltpu.SemaphoreType.DMA,
            pltpu.SemaphoreType.DMA,
            pltpu.SemaphoreType.DMA,
            pltpu.SemaphoreType.DMA,
            pltpu.SemaphoreType.DMA,
            pltpu.SemaphoreType.DMA,
        ],
    )(functools.partial(_sc_body, nrows, xdim, d))

    tgt_t = jnp.transpose(targets, (1, 2, 3, 0))
    ab = sc_call(rand_noise, u, tgt_t, logits.reshape(batch * xdim, d))

    out = pl.pallas_call(
        functools.partial(_tc_body, batch),
        out_shape=jax.ShapeDtypeStruct((1, 1), jnp.float32),
        out_specs=pl.BlockSpec(memory_space=pltpu.SMEM),
    )(ab)
    return out[0, 0]


# speedup vs baseline: 22.3639x; 1.0426x over previous
import functools

import jax
import jax.numpy as jnp
from jax import lax
from jax.experimental import pallas as pl
from jax.experimental.pallas import tpu as pltpu
from jax.experimental.pallas import tpu_sc as plsc

L = 16
NWORKERS = 32
UNROLL = 8
INT_MAX = 2147483647


def _zero_hist(hist_v, nchunk):
    zv = jnp.zeros((L,), jnp.int32)

    def z_body(g, carry):
        base = g * (L * UNROLL)
        for t in range(UNROLL):
            hist_v[pl.ds(pl.multiple_of(base + t * L, L), L)] = zv
        return carry

    lax.fori_loop(0, nchunk // UNROLL, z_body, 0)


def _select(noise_v, hist_v, k_s, lanes, lanes_seg, nchunk, seglen):
    ones = jnp.full((L,), 1, jnp.int32)
    zv = jnp.zeros((L,), jnp.int32)
    kp = jnp.int32(nchunk * L) - k_s

    def h_body(g, carry):
        base = g * (L * UNROLL)
        nvs = [noise_v[pl.ds(pl.multiple_of(base + t * L, L), L)]
               for t in range(UNROLL)]
        for nv in nvs:
            plsc.addupdate_scatter(hist_v, [nv], ones)
        return carry

    lax.fori_loop(0, nchunk // UNROLL, h_body, 0)

    seglen_c = jnp.int32(seglen)

    def s_body(g, acc):
        i0 = g * UNROLL
        for t in range(UNROLL):
            w = lanes + (i0 + t)
            w = jnp.where(w >= seglen_c, w - seglen_c, w)
            acc = acc + plsc.load_gather(hist_v, [lanes_seg + w])
        return acc

    seg = lax.fori_loop(0, seglen // UNROLL, s_body,
                        jnp.zeros((L,), jnp.int32))
    cs = plsc.cumsum(seg)
    below = cs < kp
    s_star = jnp.sum(jnp.where(below, 1, 0))
    base_cle = jnp.sum(jnp.where(below, seg, 0))

    seg_base = s_star * seglen
    chs = [hist_v[pl.ds(pl.multiple_of(seg_base + t * L, L), L)]
           for t in range(seglen // L)]
    tots = [jnp.sum(ch) for ch in chs]
    r = base_cle
    rb_star = base_cle
    t_star = jnp.int32(0)
    for t in range(seglen // L):
        nxt = r + tots[t]
        below_t = nxt < kp
        rb_star = jnp.where(below_t, nxt, rb_star)
        t_star = t_star + jnp.where(below_t, 1, 0)
        r = nxt
    ch = hist_v[pl.ds(pl.multiple_of(seg_base + t_star * L, L), L)]
    pc = plsc.cumsum(ch) + rb_star
    hitm = pc >= kp
    lane_cnt = jnp.sum(jnp.where(hitm, 0, 1))
    v_star = seg_base + t_star * L + lane_cnt
    cle_v = jnp.min(jnp.where(hitm, pc, jnp.int32(INT_MAX)))

    mp1 = cle_v - kp + 1

    def o_body(g, acc):
        i0 = g * UNROLL
        for t in range(UNROLL):
            w = lanes + (i0 + t)
            w = jnp.where(w >= seglen_c, w - seglen_c, w)
            ng = plsc.load_gather(noise_v, [lanes_seg + w])
            acc = acc + jnp.where(ng == v_star, 1, 0)
            hist_v[pl.ds(pl.multiple_of((g * UNROLL + t) * L, L), L)] = zv
        return acc

    seg2 = lax.fori_loop(0, seglen // UNROLL, o_body,
                         jnp.zeros((L,), jnp.int32))
    cs2 = plsc.cumsum(seg2)
    below2 = cs2 < mp1
    p_star = jnp.sum(jnp.where(below2, 1, 0))
    base_occ = jnp.sum(jnp.where(below2, seg2, 0))

    pos_base = p_star * seglen
    nvs = [noise_v[pl.ds(pl.multiple_of(pos_base + t * L, L), L)]
           for t in range(seglen // L)]
    eqs = [jnp.where(nv == v_star, 1, 0) for nv in nvs]
    cnts = [jnp.sum(eq) for eq in eqs]
    r2 = base_occ
    rb2_star = base_occ
    t2_star = jnp.int32(0)
    for t in range(seglen // L):
        nxt = r2 + cnts[t]
        below_t = nxt < mp1
        rb2_star = jnp.where(below_t, nxt, rb2_star)
        t2_star = t2_star + jnp.where(below_t, 1, 0)
        r2 = nxt
    nv = noise_v[pl.ds(pl.multiple_of(pos_base + t2_star * L, L), L)]
    eq = nv == v_star
    pc2 = plsc.cumsum(jnp.where(eq, 1, 0)) + rb2_star
    hit2 = eq & (pc2 == mp1)
    lane = jnp.sum(jnp.where(hit2, lanes, 0))
    return pos_base + t2_star * L + lane


def _softmax_stats(rows_v, targets_v, j_star, b, d):
    dchunk = d // L

    chs = [rows_v[0, pl.ds(pl.multiple_of(c * L, L), L)]
           for c in range(dchunk)]
    ms = chs
    while len(ms) > 1:
        ms = [jnp.maximum(ms[i], ms[i + 1]) for i in range(0, len(ms) - 1, 2)] \
            + ([ms[-1]] if len(ms) % 2 else [])
    mx = jnp.max(ms[0])
    es = [jnp.exp(ch - mx) for ch in chs]
    while len(es) > 1:
        es = [es[i] + es[i + 1] for i in range(0, len(es) - 1, 2)] \
            + ([es[-1]] if len(es) % 2 else [])
    s = jnp.sum(es[0])

    wv = jnp.full((L,), j_star & 31, dtype=jnp.int32)
    bv = jnp.full((L,), b, dtype=jnp.int32)
    tgt_vec = plsc.load_gather(targets_v, [wv, bv])
    tl_vec = plsc.load_gather(rows_v, [jnp.zeros((L,), jnp.int32), tgt_vec])
    a = jnp.max(tl_vec) - mx
    return a, s


def _sc_body(nrows, xdim, d, noise_hbm, u_hbm, targets_hbm, logits_hbm,
             out_hbm, noise_a, noise_b, u_v, hist_v, tgt_a, tgt_b,
             rows_a, rows_b, idx_a, idx_b, stage_a, stage_b, sem_n0,
             sem_n1, sem_t0, sem_t1, sem_g0, sem_g1, sem_u, sem_o0,
             sem_o1):
    nchunk = xdim // L
    seglen = xdim // L
    wid = lax.axis_index("s") * 2 + lax.axis_index("c")
    lanes = lax.iota(jnp.int32, L)
    lanes_seg = lanes * seglen
    b0 = wid * nrows
    b1 = b0 + 1

    h_n0 = pltpu.async_copy(noise_hbm.at[b0], noise_a, sem_n0)
    h_n1 = pltpu.async_copy(noise_hbm.at[b1], noise_b, sem_n1)
    h_u = pltpu.async_copy(u_hbm, u_v, sem_u)
    _zero_hist(hist_v, nchunk)
    h_u.wait()

    def get_k(b):
        b_vec = jnp.full((L,), b, dtype=jnp.int32)
        u_b = plsc.load_gather(u_v, [b_vec])
        k_vec = jnp.clip((u_b * jnp.float32(xdim)).astype(jnp.int32),
                         0, xdim - 1)
        return jnp.max(k_vec)

    k0 = get_k(b0)
    k1 = get_k(b1)

    two = lanes < 2
    idx_lanes = jnp.where(two, lanes, 0)

    h_n0.wait()
    j0 = _select(noise_a, hist_v, k0, lanes, lanes_seg, nchunk, seglen)
    plsc.store_scatter(idx_a, [idx_lanes],
                       jnp.full((L,), b0 * xdim + j0, dtype=jnp.int32),
                       mask=two)
    h_g0 = pltpu.async_copy(logits_hbm.at[idx_a], rows_a, sem_g0)
    h_t0 = pltpu.async_copy(targets_hbm.at[j0 >> 10, (j0 >> 5) & 31],
                            tgt_a, sem_t0)

    h_n1.wait()
    j1 = _select(noise_b, hist_v, k1, lanes, lanes_seg, nchunk, seglen)
    plsc.store_scatter(idx_b, [idx_lanes],
                       jnp.full((L,), b1 * xdim + j1, dtype=jnp.int32),
                       mask=two)
    h_g1 = pltpu.async_copy(logits_hbm.at[idx_b], rows_b, sem_g1)
    h_t1 = pltpu.async_copy(targets_hbm.at[j1 >> 10, (j1 >> 5) & 31],
                            tgt_b, sem_t1)

    h_g0.wait()
    h_t0.wait()
    a0, s0 = _softmax_stats(rows_a, tgt_a, j0, b0, d)
    stage_a[...] = jnp.where(lanes == 0, a0, jnp.where(lanes == 1, s0, 0.0))
    h_o0 = pltpu.async_copy(stage_a, out_hbm.at[b0], sem_o0)

    h_g1.wait()
    h_t1.wait()
    a1, s1 = _softmax_stats(rows_b, tgt_b, j1, b1, d)
    stage_b[...] = jnp.where(lanes == 0, a1, jnp.where(lanes == 1, s1, 0.0))
    h_o1 = pltpu.async_copy(stage_b, out_hbm.at[b1], sem_o1)

    h_o0.wait()
    h_o1.wait()


def _tc_body(batch, ab_ref, o_ref):
    a = ab_ref[:, 0:1]
    s = ab_ref[:, 1:2]
    ll = a - jnp.log(s)
    o_ref[0, 0] = jnp.sum(ll) * jnp.float32(1.0 / batch)


def kernel(x, logits, rand_noise, u, targets):
    batch, xdim, d = logits.shape
    nrows = batch // NWORKERS
    mesh = plsc.VectorSubcoreMesh(core_axis_name="c", subcore_axis_name="s")
    sc_call = functools.partial(
        pl.kernel,
        out_type=jax.ShapeDtypeStruct((batch, L), jnp.float32),
        mesh=mesh,
        compiler_params=pltpu.CompilerParams(needs_layout_passes=False),
        scratch_types=[
            pltpu.VMEM((xdim,), jnp.int32),
            pltpu.VMEM((xdim,), jnp.int32),
            pltpu.VMEM((batch,), jnp.float32),
            pltpu.VMEM((xdim,), jnp.int32),
            pltpu.VMEM((32, batch), jnp.int32),
            pltpu.VMEM((32, batch), jnp.int32),
            pltpu.VMEM((2, d), jnp.float32),
            pltpu.VMEM((2, d), jnp.float32),
            pltpu.VMEM((2,), jnp.int32),
            pltpu.VMEM((2,), jnp.int32),
            pltpu.VMEM((L,), jnp.float32),
            pltpu.VMEM((L,), jnp.float32),
            pltpu.SemaphoreType.DMA,
            pltpu.SemaphoreType.DMA,
            pltpu.SemaphoreType.DMA,
            pltpu.SemaphoreType.DMA,
            pltpu.SemaphoreType.DMA,
            pltpu.SemaphoreType.DMA,
            pltpu.SemaphoreType.DMA,
            pltpu.SemaphoreType.DMA,
            pltpu.SemaphoreType.DMA,
        ],
    )(functools.partial(_sc_body, nrows, xdim, d))

    tgt_t = jnp.transpose(targets, (1, 2, 3, 0))
    ab = sc_call(rand_noise, u, tgt_t, logits.reshape(batch * xdim, d))

    out = pl.pallas_call(
        functools.partial(_tc_body, batch),
        out_shape=jax.ShapeDtypeStruct((1, 1), jnp.float32),
        out_specs=pl.BlockSpec(memory_space=pltpu.SMEM),
    )(ab)
    return out[0, 0]
